# Initial kernel scaffold; baseline (speedup 1.0000x reference)
import jax
import jax.numpy as jnp
from jax.experimental import pallas as pl


def kernel(class_logits, box_regression, proposals):
    raise NotImplementedError("write your pallas kernel here")


# speedup vs baseline: 123.2160x; 123.2160x over previous
import math

import jax
import jax.numpy as jnp
from jax.experimental import pallas as pl
from jax.experimental.pallas import tpu as pltpu

_SCORE_T = 0.3
_NMS_T = 0.3
_DET = 320
_IMG = 800.0
_MINSZ = 0.01
_CLIP = math.log(1000.0 / 16.0)
_N = 5000
_NPAD = 5120
_M = 10000
_B = 256
_NB = 40


def _decode_kernel(lg_ref, br_ref, p_ref, x1_ref, y1_ref, x2_ref, y2_ref,
                   sc_ref, val_ref):
    p = p_ref[:, :]
    w = p[2:3] - p[0:1]
    h = p[3:4] - p[1:2]
    cx = p[0:1] + 0.5 * w
    cy = p[1:2] + 0.5 * h
    lg = lg_ref[:, :]
    mx = jnp.max(lg, axis=0, keepdims=True)
    e = jnp.exp(lg - mx)
    probs = e / jnp.sum(e, axis=0, keepdims=True)
    br = br_ref[:, :]
    col = jax.lax.broadcasted_iota(jnp.int32, (1, _NPAD), 1)
    in_range = col < _N
    for c in (1, 2):
        dx = br[4 * c:4 * c + 1] * (1.0 / 10.0)
        dy = br[4 * c + 1:4 * c + 2] * (1.0 / 10.0)
        dw = jnp.minimum(br[4 * c + 2:4 * c + 3] * (1.0 / 5.0), _CLIP)
        dh = jnp.minimum(br[4 * c + 3:4 * c + 4] * (1.0 / 5.0), _CLIP)
        pcx = dx * w + cx
        pcy = dy * h + cy
        pw = jnp.exp(dw) * w
        ph = jnp.exp(dh) * h
        x1 = jnp.clip(pcx - 0.5 * pw, 0.0, _IMG)
        y1 = jnp.clip(pcy - 0.5 * ph, 0.0, _IMG)
        x2 = jnp.clip(pcx + 0.5 * pw, 0.0, _IMG)
        y2 = jnp.clip(pcy + 0.5 * ph, 0.0, _IMG)
        sc = probs[c:c + 1]
        valid = ((sc > _SCORE_T) & (x2 - x1 >= _MINSZ) & (y2 - y1 >= _MINSZ)
                 & in_range)
        r = c - 1
        x1_ref[r:r + 1, :] = x1
        y1_ref[r:r + 1, :] = y1
        x2_ref[r:r + 1, :] = x2
        y2_ref[r:r + 1, :] = y2
        sc_ref[r:r + 1, :] = sc
        val_ref[r:r + 1, :] = jnp.where(valid, 1.0, 0.0)


def _nms_kernel(x1_ref, y1_ref, x2_ref, y2_ref, lbl_ref, val_ref, keep_ref,
                sx1, sy1, sx2, sy2, sar, smask):
    off = lbl_ref[:, :] * (_IMG + 1.0)
    x1 = x1_ref[:, :]
    y1 = y1_ref[:, :]
    x2 = x2_ref[:, :]
    y2 = y2_ref[:, :]
    sx1[:, :] = x1 + off
    sy1[:, :] = y1 + off
    sx2[:, :] = x2 + off
    sy2[:, :] = y2 + off
    sar[:, :] = (x2 - x1) * (y2 - y1)
    val = val_ref[:, :]
    keep_ref[:, :] = val
    nvalid = jnp.sum(val).astype(jnp.int32)
    nbv = jnp.minimum((nvalid + _B - 1) // _B, _NB)

    colid = jax.lax.broadcasted_iota(jnp.int32, (1, _B), 1)
    rowi = jax.lax.broadcasted_iota(jnp.int32, (_B, _B), 0)
    colj = jax.lax.broadcasted_iota(jnp.int32, (_B, _B), 1)

    def blk(bi, carry):
        ax1 = sx1[pl.ds(bi, 1), :]
        ay1 = sy1[pl.ds(bi, 1), :]
        ax2 = sx2[pl.ds(bi, 1), :]
        ay2 = sy2[pl.ds(bi, 1), :]
        aar = sar[pl.ds(bi, 1), :]
        cx1 = ax1.reshape(_B, 1)
        cy1 = ay1.reshape(_B, 1)
        cx2 = ax2.reshape(_B, 1)
        cy2 = ay2.reshape(_B, 1)
        car = aar.reshape(_B, 1)
        xx1 = jnp.maximum(cx1, ax1)
        yy1 = jnp.maximum(cy1, ay1)
        xx2 = jnp.minimum(cx2, ax2)
        yy2 = jnp.minimum(cy2, ay2)
        inter = jnp.maximum(xx2 - xx1, 0.0) * jnp.maximum(yy2 - yy1, 0.0)
        iou = inter / (car + aar - inter + 1e-9)
        smask[:, :] = jnp.where((iou > _NMS_T) & (colj > rowi), 1.0, 0.0)

        def step(k, kvf):
            row = smask[pl.ds(k, 1), :]
            kept_k = jnp.sum(jnp.where(colid == k, kvf, 0.0))
            supp = jnp.where(kept_k > 0.0, row, 0.0)
            return kvf * (1.0 - supp)

        kvf = jax.lax.fori_loop(0, _B, step, keep_ref[pl.ds(bi, 1), :])
        keep_ref[pl.ds(bi, 1), :] = kvf
        kcol = kvf.reshape(_B, 1)

        def cross(bj, c2):
            bx1 = sx1[pl.ds(bj, 1), :]
            by1 = sy1[pl.ds(bj, 1), :]
            bx2 = sx2[pl.ds(bj, 1), :]
            by2 = sy2[pl.ds(bj, 1), :]
            bar = sar[pl.ds(bj, 1), :]
            u1 = jnp.maximum(cx1, bx1)
            v1 = jnp.maximum(cy1, by1)
            u2 = jnp.minimum(cx2, bx2)
            v2 = jnp.minimum(cy2, by2)
            it = jnp.maximum(u2 - u1, 0.0) * jnp.maximum(v2 - v1, 0.0)
            io = it / (car + bar - it + 1e-9)
            hit = jnp.where(io > _NMS_T, kcol, 0.0)
            anyhit = jnp.max(hit, axis=0, keepdims=True)
            keep_ref[pl.ds(bj, 1), :] = keep_ref[pl.ds(bj, 1), :] * (1.0 - anyhit)
            return c2

        jax.lax.fori_loop(bi + 1, nbv, cross, 0)
        return carry

    jax.lax.fori_loop(0, nbv, blk, 0)


def kernel(class_logits, box_regression, proposals):
    padn = _NPAD - _N
    lgT = jnp.pad(class_logits.T, ((0, 0), (0, padn)))
    brT = jnp.pad(box_regression.T, ((0, 0), (0, padn)))
    pT = jnp.pad(proposals.T, ((0, 0), (0, padn)))

    shp = jax.ShapeDtypeStruct((2, _NPAD), jnp.float32)
    x1, y1, x2, y2, sc, val = pl.pallas_call(
        _decode_kernel,
        out_shape=(shp, shp, shp, shp, shp, shp),
        interpret=False,
    )(lgT, brT, pT)

    def flat(a):
        return a[:, :_N].T.reshape(-1)

    fx1, fy1, fx2, fy2 = flat(x1), flat(y1), flat(x2), flat(y2)
    fsc, fval = flat(sc), flat(val) > 0.5
    labels = (jnp.arange(_M, dtype=jnp.int32) % 2) + 1

    sort_scores = jnp.where(fval, fsc, -1.0)
    order = jnp.argsort(-sort_scores)

    padm = _NB * _B - _M

    def sortpad(a, fill=0.0):
        return jnp.pad(a[order], ((0, padm),), constant_values=fill)

    X1 = sortpad(fx1).reshape(_NB, _B)
    Y1 = sortpad(fy1).reshape(_NB, _B)
    X2 = sortpad(fx2).reshape(_NB, _B)
    Y2 = sortpad(fy2).reshape(_NB, _B)
    LBL = sortpad(labels.astype(jnp.float32)).reshape(_NB, _B)
    VAL = sortpad(jnp.where(fval, 1.0, 0.0)).reshape(_NB, _B)
    sc_s = fsc[order]
    lab_s = labels[order]

    keep2 = pl.pallas_call(
        _nms_kernel,
        out_shape=jax.ShapeDtypeStruct((_NB, _B), jnp.float32),
        scratch_shapes=[
            pltpu.VMEM((_NB, _B), jnp.float32),
            pltpu.VMEM((_NB, _B), jnp.float32),
            pltpu.VMEM((_NB, _B), jnp.float32),
            pltpu.VMEM((_NB, _B), jnp.float32),
            pltpu.VMEM((_NB, _B), jnp.float32),
            pltpu.VMEM((_B, _B), jnp.float32),
        ],
        interpret=False,
    )(X1, Y1, X2, Y2, LBL, VAL)
    keep = keep2.reshape(-1)[:_M] > 0.5

    rank_key = jnp.where(keep,
                         jnp.where(lab_s == 2, 10.0, 0.0) + sc_s,
                         -1e9)
    _, top_idx = jax.lax.top_k(rank_key, _DET)
    fvalid = keep[top_idx]
    boxes_s = jnp.stack([fx1[order], fy1[order], fx2[order], fy2[order]],
                        axis=1)
    out_boxes = jnp.where(fvalid[:, None], boxes_s[top_idx], 0.0)
    out_scores = jnp.where(fvalid, sc_s[top_idx], 0.0)
    out_labels = jnp.where(fvalid, lab_s[top_idx], 0).astype(jnp.int32)
    return out_boxes, out_scores, out_labels


# speedup vs baseline: 658.9061x; 5.3476x over previous
import functools
import math

import jax
import jax.numpy as jnp
from jax import lax
from jax.experimental import pallas as pl
from jax.experimental.pallas import tpu as pltpu
from jax.experimental.pallas import tpu_sc as plsc

_SCORE_T = 0.3
_NMS_T = 0.3
_DET = 320
_IMG = 800.0
_MINSZ = 0.01
_CLIP = math.log(1000.0 / 16.0)
_N = 5000
_NPAD = 5120
_M = 10000
_B = 256
_NB = 40


def _decode_kernel(lg_ref, br_ref, p_ref, x1_ref, y1_ref, x2_ref, y2_ref,
                   sc_ref, val_ref):
    p = p_ref[:, :]
    w = p[2:3] - p[0:1]
    h = p[3:4] - p[1:2]
    cx = p[0:1] + 0.5 * w
    cy = p[1:2] + 0.5 * h
    lg = lg_ref[:, :]
    mx = jnp.max(lg, axis=0, keepdims=True)
    e = jnp.exp(lg - mx)
    probs = e / jnp.sum(e, axis=0, keepdims=True)
    br = br_ref[:, :]
    col = jax.lax.broadcasted_iota(jnp.int32, (1, _NPAD), 1)
    in_range = col < _N
    for c in (1, 2):
        dx = br[4 * c:4 * c + 1] * (1.0 / 10.0)
        dy = br[4 * c + 1:4 * c + 2] * (1.0 / 10.0)
        dw = jnp.minimum(br[4 * c + 2:4 * c + 3] * (1.0 / 5.0), _CLIP)
        dh = jnp.minimum(br[4 * c + 3:4 * c + 4] * (1.0 / 5.0), _CLIP)
        pcx = dx * w + cx
        pcy = dy * h + cy
        pw = jnp.exp(dw) * w
        ph = jnp.exp(dh) * h
        x1 = jnp.clip(pcx - 0.5 * pw, 0.0, _IMG)
        y1 = jnp.clip(pcy - 0.5 * ph, 0.0, _IMG)
        x2 = jnp.clip(pcx + 0.5 * pw, 0.0, _IMG)
        y2 = jnp.clip(pcy + 0.5 * ph, 0.0, _IMG)
        sc = probs[c:c + 1]
        valid = ((sc > _SCORE_T) & (x2 - x1 >= _MINSZ) & (y2 - y1 >= _MINSZ)
                 & in_range)
        r = c - 1
        x1_ref[r:r + 1, :] = x1
        y1_ref[r:r + 1, :] = y1
        x2_ref[r:r + 1, :] = x2
        y2_ref[r:r + 1, :] = y2
        sc_ref[r:r + 1, :] = sc
        val_ref[r:r + 1, :] = jnp.where(valid, 1.0, 0.0)


def _nms_kernel(x1_ref, y1_ref, x2_ref, y2_ref, lbl_ref, val_ref, keep_ref,
                sx1, sy1, sx2, sy2, sar, smask):
    off = lbl_ref[:, :] * (_IMG + 1.0)
    x1 = x1_ref[:, :]
    y1 = y1_ref[:, :]
    x2 = x2_ref[:, :]
    y2 = y2_ref[:, :]
    sx1[:, :] = x1 + off
    sy1[:, :] = y1 + off
    sx2[:, :] = x2 + off
    sy2[:, :] = y2 + off
    sar[:, :] = (x2 - x1) * (y2 - y1)
    val = val_ref[:, :]
    keep_ref[:, :] = val
    nvalid = jnp.sum(val).astype(jnp.int32)
    nbv = jnp.minimum((nvalid + _B - 1) // _B, _NB)

    colid = jax.lax.broadcasted_iota(jnp.int32, (1, _B), 1)
    rowi = jax.lax.broadcasted_iota(jnp.int32, (_B, _B), 0)
    colj = jax.lax.broadcasted_iota(jnp.int32, (_B, _B), 1)

    def blk(bi, carry):
        ax1 = sx1[pl.ds(bi, 1), :]
        ay1 = sy1[pl.ds(bi, 1), :]
        ax2 = sx2[pl.ds(bi, 1), :]
        ay2 = sy2[pl.ds(bi, 1), :]
        aar = sar[pl.ds(bi, 1), :]
        cx1 = ax1.reshape(_B, 1)
        cy1 = ay1.reshape(_B, 1)
        cx2 = ax2.reshape(_B, 1)
        cy2 = ay2.reshape(_B, 1)
        car = aar.reshape(_B, 1)
        xx1 = jnp.maximum(cx1, ax1)
        yy1 = jnp.maximum(cy1, ay1)
        xx2 = jnp.minimum(cx2, ax2)
        yy2 = jnp.minimum(cy2, ay2)
        inter = jnp.maximum(xx2 - xx1, 0.0) * jnp.maximum(yy2 - yy1, 0.0)
        iou = inter / (car + aar - inter + 1e-9)
        smask[:, :] = jnp.where((iou > _NMS_T) & (colj > rowi), 1.0, 0.0)

        und0 = keep_ref[pl.ds(bi, 1), :]

        def fcond(state):
            _, und = state
            return jnp.sum(und) > 0.0

        def fbody(state):
            kept, und = state
            m = smask[:, :]
            alive_col = (kept + und).reshape(_B, 1)
            threat_alive = jnp.max(m * alive_col, axis=0, keepdims=True)
            new_kept = und * (1.0 - threat_alive)
            kept2 = kept + new_kept
            threat_kept = jnp.max(m * kept2.reshape(_B, 1), axis=0,
                                  keepdims=True)
            und2 = und * (1.0 - new_kept) * (1.0 - threat_kept)
            return kept2, und2

        kvf, _ = jax.lax.while_loop(fcond, fbody,
                                    (jnp.zeros_like(und0), und0))
        keep_ref[pl.ds(bi, 1), :] = kvf
        kcol = kvf.reshape(_B, 1)

        def cross(bj, c2):
            bx1 = sx1[pl.ds(bj, 1), :]
            by1 = sy1[pl.ds(bj, 1), :]
            bx2 = sx2[pl.ds(bj, 1), :]
            by2 = sy2[pl.ds(bj, 1), :]
            bar = sar[pl.ds(bj, 1), :]
            u1 = jnp.maximum(cx1, bx1)
            v1 = jnp.maximum(cy1, by1)
            u2 = jnp.minimum(cx2, bx2)
            v2 = jnp.minimum(cy2, by2)
            it = jnp.maximum(u2 - u1, 0.0) * jnp.maximum(v2 - v1, 0.0)
            io = it / (car + bar - it + 1e-9)
            hit = jnp.where(io > _NMS_T, kcol, 0.0)
            anyhit = jnp.max(hit, axis=0, keepdims=True)
            keep_ref[pl.ds(bj, 1), :] = keep_ref[pl.ds(bj, 1), :] * (1.0 - anyhit)
            return c2

        jax.lax.fori_loop(bi + 1, nbv, cross, 0)
        return carry

    jax.lax.fori_loop(0, nbv, blk, 0)


_MT = _NB * _B
_TBL_C = 7
_NC = 2
_NS = 16
_NW = _NC * _NS
_GT = _TBL_C * _MT
_BPW = _GT // _NW


@functools.partial(
    pl.kernel,
    mesh=plsc.VectorSubcoreMesh(core_axis_name="c", subcore_axis_name="s"),
    out_type=jax.ShapeDtypeStruct((_GT,), jnp.float32),
    scratch_types=[
        pltpu.VMEM((_BPW,), jnp.int32),
        pltpu.VMEM((_BPW,), jnp.float32),
        pltpu.SemaphoreType.DMA,
    ],
)
def _sc_gather(tbl_hbm, idx_hbm, out_hbm, idx_v, rows_v, sem):
    wid = lax.axis_index("s") * _NC + lax.axis_index("c")
    base = wid * _BPW
    pltpu.sync_copy(idx_hbm.at[pl.ds(base, _BPW)], idx_v)
    pltpu.async_copy(tbl_hbm.at[idx_v], rows_v, sem).wait()
    pltpu.sync_copy(rows_v, out_hbm.at[pl.ds(base, _BPW)])


def kernel(class_logits, box_regression, proposals):
    padn = _NPAD - _N
    lgT = jnp.pad(class_logits.T, ((0, 0), (0, padn)))
    brT = jnp.pad(box_regression.T, ((0, 0), (0, padn)))
    pT = jnp.pad(proposals.T, ((0, 0), (0, padn)))

    shp = jax.ShapeDtypeStruct((2, _NPAD), jnp.float32)
    x1, y1, x2, y2, sc, val = pl.pallas_call(
        _decode_kernel,
        out_shape=(shp, shp, shp, shp, shp, shp),
        interpret=False,
    )(lgT, brT, pT)

    def flat(a):
        return a[:, :_N].T.reshape(-1)

    fx1, fy1, fx2, fy2 = flat(x1), flat(y1), flat(x2), flat(y2)
    fsc, fval = flat(sc), flat(val) > 0.5
    labels = (jnp.arange(_M, dtype=jnp.int32) % 2) + 1

    sort_scores = jnp.where(fval, fsc, -1.0)
    order = jnp.argsort(-sort_scores)

    def padm(a):
        return jnp.pad(a, ((0, _MT - _M),))

    tbl = jnp.concatenate([padm(fx1), padm(fy1), padm(fx2), padm(fy2),
                           padm(fsc), padm(jnp.where(fval, 1.0, 0.0)),
                           padm(labels.astype(jnp.float32))])
    idx1 = jnp.pad(order.astype(jnp.int32), ((0, _MT - _M),),
                   constant_values=_M)
    idx = (idx1[None, :]
           + (jnp.arange(_TBL_C, dtype=jnp.int32) * _MT)[:, None]).reshape(-1)
    st = _sc_gather(tbl, idx).reshape(_TBL_C, _MT)

    X1 = st[0].reshape(_NB, _B)
    Y1 = st[1].reshape(_NB, _B)
    X2 = st[2].reshape(_NB, _B)
    Y2 = st[3].reshape(_NB, _B)
    VAL = st[5].reshape(_NB, _B)
    LBL = st[6].reshape(_NB, _B)
    sc_s = st[4, :_M]
    lab_s = st[6, :_M].astype(jnp.int32)

    keep2 = pl.pallas_call(
        _nms_kernel,
        out_shape=jax.ShapeDtypeStruct((_NB, _B), jnp.float32),
        scratch_shapes=[
            pltpu.VMEM((_NB, _B), jnp.float32),
            pltpu.VMEM((_NB, _B), jnp.float32),
            pltpu.VMEM((_NB, _B), jnp.float32),
            pltpu.VMEM((_NB, _B), jnp.float32),
            pltpu.VMEM((_NB, _B), jnp.float32),
            pltpu.VMEM((_B, _B), jnp.float32),
        ],
        interpret=False,
    )(X1, Y1, X2, Y2, LBL, VAL)
    keep = keep2.reshape(-1)[:_M] > 0.5

    rank_key = jnp.where(keep,
                         jnp.where(lab_s == 2, 10.0, 0.0) + sc_s,
                         -1e9)
    _, top_idx = jax.lax.top_k(rank_key, _DET)
    fvalid = keep[top_idx]
    boxes_s = st[:4, :_M].T
    out_boxes = jnp.where(fvalid[:, None], boxes_s[top_idx], 0.0)
    out_scores = jnp.where(fvalid, sc_s[top_idx], 0.0)
    out_labels = jnp.where(fvalid, lab_s[top_idx], 0).astype(jnp.int32)
    return out_boxes, out_scores, out_labels
